# Initial kernel scaffold; baseline (speedup 1.0000x reference)
import jax
import jax.numpy as jnp
from jax.experimental import pallas as pl


def kernel(feature, typeEmbed, tableEmbed, columnEmbed, opEmbed, joinEmbed, Wf, bf, Wf2, bf2, Ws, bs, Wh, bh, Wp, bp):
    raise NotImplementedError("write your pallas kernel here")


# speedup vs baseline: 3.7402x; 3.7402x over previous
import functools

import jax
import jax.numpy as jnp
from jax.experimental import pallas as pl
from jax.experimental.pallas import tpu as pltpu

ES = 32
BIN = 50
FD = ES + ES // 8 + 1
PD = 5 * ES + ES // 8 + 1
FEAT_DIM = 1 + 1 + 9 + 3 + BIN * 3 + 1001

TILE = 512


def _leaky(x):
    return jnp.where(x >= 0, x, 0.01 * x)


def _body(f_ref, a102_ref, v3_ref, b1_ref, w2_ref, b2_ref, m1_ref, bh_ref,
          g70_ref, wp2_ref, wp4_ref, wp5_ref, bp_ref, o_ref):
    f = f_ref[...]
    t = f.shape[0]

    type_id = f[:, 0:1].astype(jnp.int32)
    join_id = f[:, 1:2].astype(jnp.int32)
    cols = f[:, 2:5].astype(jnp.int32)
    ops = f[:, 5:8].astype(jnp.int32)
    vals = f[:, 8:11]
    m = f[:, 11:14]
    tab_id = f[:, 164:165].astype(jnp.int32)

    i102 = jax.lax.broadcasted_iota(jnp.int32, (t, 102), 1)
    oh102 = (
        (i102 == cols[:, 0:1]).astype(jnp.float32)
        + (i102 == cols[:, 1:2] + 30).astype(jnp.float32)
        + (i102 == cols[:, 2:3] + 60).astype(jnp.float32)
        + (i102 == ops[:, 0:1] + 90).astype(jnp.float32)
        + (i102 == ops[:, 1:2] + 94).astype(jnp.float32)
        + (i102 == ops[:, 2:3] + 98).astype(jnp.float32)
    )
    h1 = (jnp.dot(oh102, a102_ref[...], preferred_element_type=jnp.float32)
          + jnp.dot(vals, v3_ref[...], preferred_element_type=jnp.float32)
          + b1_ref[...])
    h1 = _leaky(h1)
    h2 = _leaky(jnp.dot(h1, w2_ref[...], preferred_element_type=jnp.float32)
                + b2_ref[...])

    nf = m[:, 0:1] + m[:, 1:2] + m[:, 2:3]
    zero = jnp.zeros_like(h2[:, 0:FD])
    fsum = (jnp.where(m[:, 0:1] != 0, h2[:, 0:FD], zero)
            + jnp.where(m[:, 1:2] != 0, h2[:, FD:2 * FD], zero)
            + jnp.where(m[:, 2:3] != 0, h2[:, 2 * FD:3 * FD], zero))
    filter_emb = fsum / nf

    hs = jnp.dot(f, m1_ref[...], preferred_element_type=jnp.float32)
    bh = bh_ref[...]
    zero32 = jnp.zeros_like(hs[:, 0:ES])
    hist_sum = (jnp.where(m[:, 0:1] != 0, hs[:, 0:ES] + bh, zero32)
                + jnp.where(m[:, 1:2] != 0, hs[:, ES:2 * ES] + bh, zero32)
                + jnp.where(m[:, 2:3] != 0, hs[:, 2 * ES:3 * ES] + bh, zero32))
    hist_emb = hist_sum / nf
    samp = hs[:, 96:128]

    i70 = jax.lax.broadcasted_iota(jnp.int32, (t, 70), 1)
    oh70 = ((i70 == type_id).astype(jnp.float32)
            + (i70 == join_id + 20).astype(jnp.float32)
            + (i70 == tab_id + 60).astype(jnp.float32))
    pre = (jnp.dot(oh70, g70_ref[...], preferred_element_type=jnp.float32)
           + jnp.dot(filter_emb, wp2_ref[...], preferred_element_type=jnp.float32)
           + jnp.dot(samp, wp4_ref[...], preferred_element_type=jnp.float32)
           + jnp.dot(hist_emb, wp5_ref[...], preferred_element_type=jnp.float32)
           + bp_ref[...])
    o_ref[...] = _leaky(pre)


@functools.partial(jax.jit, static_argnames=())
def kernel(feature, typeEmbed, tableEmbed, columnEmbed, opEmbed, joinEmbed,
           Wf, bf, Wf2, bf2, Ws, bs, Wh, bh, Wp, bp):
    b = feature.shape[0]
    f32 = jnp.float32

    a_col = jnp.dot(columnEmbed, Wf[:, :ES].T)
    a_op = jnp.dot(opEmbed, Wf[:, ES:ES + 4].T)
    a_val = Wf[:, ES + 4]
    a102 = jnp.zeros((102, 3 * FD), f32)
    a102 = a102.at[0:30, 0:FD].set(a_col)
    a102 = a102.at[30:60, FD:2 * FD].set(a_col)
    a102 = a102.at[60:90, 2 * FD:3 * FD].set(a_col)
    a102 = a102.at[90:94, 0:FD].set(a_op)
    a102 = a102.at[94:98, FD:2 * FD].set(a_op)
    a102 = a102.at[98:102, 2 * FD:3 * FD].set(a_op)
    v3 = jnp.zeros((3, 3 * FD), f32)
    v3 = v3.at[0, 0:FD].set(a_val)
    v3 = v3.at[1, FD:2 * FD].set(a_val)
    v3 = v3.at[2, 2 * FD:3 * FD].set(a_val)
    b1 = jnp.tile(bf, 3)[None]
    w2 = jnp.zeros((3 * FD, 3 * FD), f32)
    w2 = w2.at[0:FD, 0:FD].set(Wf2.T)
    w2 = w2.at[FD:2 * FD, FD:2 * FD].set(Wf2.T)
    w2 = w2.at[2 * FD:3 * FD, 2 * FD:3 * FD].set(Wf2.T)
    b2 = jnp.tile(bf2, 3)[None]

    mh = jnp.zeros((BIN, 3, 3, ES), f32)
    wht = Wh.T
    mh = mh.at[:, 0, 0, :].set(wht)
    mh = mh.at[:, 1, 1, :].set(wht)
    mh = mh.at[:, 2, 2, :].set(wht)
    m1 = jnp.zeros((FEAT_DIM, 128), f32)
    m1 = m1.at[14:164, 0:96].set(mh.reshape(3 * BIN, 3 * ES))
    m1 = m1.at[165:FEAT_DIM, 96:128].set(Ws.T)

    wpt = Wp.T
    g70 = jnp.concatenate([
        jnp.dot(typeEmbed, wpt[0:ES]),
        jnp.dot(joinEmbed, wpt[ES + FD:2 * ES + FD]),
        jnp.dot(tableEmbed, wpt[2 * ES + FD:3 * ES + FD]),
    ], axis=0)
    wp2 = wpt[ES:ES + FD]
    wp4 = wpt[2 * ES + FD:3 * ES + FD]
    wp5 = wpt[PD - ES:PD]
    bp2 = (bp + jnp.dot(bs, wp4))[None]
    bh2 = bh[None]

    grid = (b // TILE,)
    full = lambda s: pl.BlockSpec(s, lambda i: (0,) * len(s))
    out = pl.pallas_call(
        _body,
        grid=grid,
        in_specs=[
            pl.BlockSpec((TILE, FEAT_DIM), lambda i: (i, 0)),
            full(a102.shape), full(v3.shape), full(b1.shape),
            full(w2.shape), full(b2.shape), full(m1.shape), full(bh2.shape),
            full(g70.shape), full(wp2.shape), full(wp4.shape),
            full(wp5.shape), full(bp2.shape),
        ],
        out_specs=pl.BlockSpec((TILE, PD), lambda i: (i, 0)),
        out_shape=jax.ShapeDtypeStruct((b, PD), f32),
        compiler_params=pltpu.CompilerParams(
            dimension_semantics=("arbitrary",),
        ),
    )(feature, a102, v3, b1, w2, b2, m1, bh2, g70, wp2, wp4, wp5, bp2)
    return out


# speedup vs baseline: 4.2717x; 1.1421x over previous
import functools

import jax
import jax.numpy as jnp
from jax.experimental import pallas as pl
from jax.experimental.pallas import tpu as pltpu

ES = 32
BIN = 50
FD = ES + ES // 8 + 1
PD = 5 * ES + ES // 8 + 1
FEAT_DIM = 1 + 1 + 9 + 3 + BIN * 3 + 1001

TILE = 512


def _leaky(x):
    return jnp.where(x >= 0, x, 0.01 * x)


def _body(f_ref, a102_ref, v3_ref, b1_ref, w2_ref, b2_ref, m1_ref, bh_ref,
          g70_ref, wp2_ref, wp4_ref, wp5_ref, bp_ref, o_ref):
    f = f_ref[...]
    t = f.shape[0]

    type_id = f[:, 0:1].astype(jnp.int32)
    join_id = f[:, 1:2].astype(jnp.int32)
    cols = f[:, 2:5].astype(jnp.int32)
    ops = f[:, 5:8].astype(jnp.int32)
    vals = f[:, 8:11]
    m = f[:, 11:14]
    tab_id = f[:, 164:165].astype(jnp.int32)

    i102 = jax.lax.broadcasted_iota(jnp.int32, (t, 102), 1)
    oh102 = (
        (i102 == cols[:, 0:1]).astype(jnp.float32)
        + (i102 == cols[:, 1:2] + 30).astype(jnp.float32)
        + (i102 == cols[:, 2:3] + 60).astype(jnp.float32)
        + (i102 == ops[:, 0:1] + 90).astype(jnp.float32)
        + (i102 == ops[:, 1:2] + 94).astype(jnp.float32)
        + (i102 == ops[:, 2:3] + 98).astype(jnp.float32)
    )
    h1 = (jnp.dot(oh102, a102_ref[...], preferred_element_type=jnp.float32)
          + jnp.dot(vals, v3_ref[...], preferred_element_type=jnp.float32)
          + b1_ref[...])
    h1 = _leaky(h1)
    h2 = _leaky(jnp.dot(h1, w2_ref[...], preferred_element_type=jnp.float32)
                + b2_ref[...])

    nf = m[:, 0:1] + m[:, 1:2] + m[:, 2:3]
    zero = jnp.zeros_like(h2[:, 0:FD])
    fsum = (jnp.where(m[:, 0:1] != 0, h2[:, 0:FD], zero)
            + jnp.where(m[:, 1:2] != 0, h2[:, FD:2 * FD], zero)
            + jnp.where(m[:, 2:3] != 0, h2[:, 2 * FD:3 * FD], zero))
    filter_emb = fsum / nf

    hs = jnp.dot(f, m1_ref[...], preferred_element_type=jnp.float32)
    bh = bh_ref[...]
    zero32 = jnp.zeros_like(hs[:, 0:ES])
    hist_sum = (jnp.where(m[:, 0:1] != 0, hs[:, 0:ES] + bh, zero32)
                + jnp.where(m[:, 1:2] != 0, hs[:, ES:2 * ES] + bh, zero32)
                + jnp.where(m[:, 2:3] != 0, hs[:, 2 * ES:3 * ES] + bh, zero32))
    hist_emb = hist_sum / nf
    samp = hs[:, 96:128]

    i70 = jax.lax.broadcasted_iota(jnp.int32, (t, 70), 1)
    oh70 = ((i70 == type_id).astype(jnp.float32)
            + (i70 == join_id + 20).astype(jnp.float32)
            + (i70 == tab_id + 60).astype(jnp.float32))
    pre = (jnp.dot(oh70, g70_ref[...], preferred_element_type=jnp.float32)
           + jnp.dot(filter_emb, wp2_ref[...], preferred_element_type=jnp.float32)
           + jnp.dot(samp, wp4_ref[...], preferred_element_type=jnp.float32)
           + jnp.dot(hist_emb, wp5_ref[...], preferred_element_type=jnp.float32)
           + bp_ref[...])
    o_ref[...] = _leaky(pre)


def _dn(a, b):
    return jax.lax.dot_general(a, b, (((1,), (1,)), ((), ())),
                               preferred_element_type=jnp.float32)


def _prep_body(te_ref, tab_ref, ce_ref, oe_ref, je_ref, wf_ref, bf_ref,
               wf2_ref, bf2_ref, ws_ref, bs_ref, wh_ref, wp_ref, bp_ref,
               a102_ref, v3_ref, b1_ref, w2_ref, b2_ref, m1_ref, g70_ref,
               wp2_ref, wp4_ref, wp5_ref, bp2_ref):
    f32 = jnp.float32
    wf = wf_ref[...]
    bf = bf_ref[...]
    bf2 = bf2_ref[...]
    wp = wp_ref[...]

    a_col = _dn(ce_ref[...], wf[:, 0:ES])
    a_op = _dn(oe_ref[...], wf[:, ES:ES + 4])
    e36 = (jax.lax.broadcasted_iota(jnp.int32, (1, FD), 1) == FD - 1)
    a_val = _dn(e36.astype(f32), wf)
    a102_ref[...] = jnp.zeros((102, 3 * FD), f32)
    a102_ref[0:30, 0:FD] = a_col
    a102_ref[30:60, FD:2 * FD] = a_col
    a102_ref[60:90, 2 * FD:3 * FD] = a_col
    a102_ref[90:94, 0:FD] = a_op
    a102_ref[94:98, FD:2 * FD] = a_op
    a102_ref[98:102, 2 * FD:3 * FD] = a_op
    v3_ref[...] = jnp.zeros((3, 3 * FD), f32)
    v3_ref[0:1, 0:FD] = a_val
    v3_ref[1:2, FD:2 * FD] = a_val
    v3_ref[2:3, 2 * FD:3 * FD] = a_val
    b1_ref[0:1, 0:FD] = bf
    b1_ref[0:1, FD:2 * FD] = bf
    b1_ref[0:1, 2 * FD:3 * FD] = bf

    i37r = jax.lax.broadcasted_iota(jnp.int32, (FD, FD), 0)
    i37c = jax.lax.broadcasted_iota(jnp.int32, (FD, FD), 1)
    wf2t = _dn((i37r == i37c).astype(f32), wf2_ref[...])
    w2_ref[...] = jnp.zeros((3 * FD, 3 * FD), f32)
    w2_ref[0:FD, 0:FD] = wf2t
    w2_ref[FD:2 * FD, FD:2 * FD] = wf2t
    w2_ref[2 * FD:3 * FD, 2 * FD:3 * FD] = wf2t
    b2_ref[0:1, 0:FD] = bf2
    b2_ref[0:1, FD:2 * FD] = bf2
    b2_ref[0:1, 2 * FD:3 * FD] = bf2

    m1_ref[...] = jnp.zeros((FEAT_DIM, 128), f32)
    rr = jax.lax.broadcasted_iota(jnp.int32, (3 * BIN, BIN), 0)
    cc = jax.lax.broadcasted_iota(jnp.int32, (3 * BIN, BIN), 1)
    wh = wh_ref[...]
    for j in range(3):
        ej = (rr == 3 * cc + j).astype(f32)
        m1_ref[14:164, ES * j:ES * (j + 1)] = _dn(ej, wh)
    ik_r = jax.lax.broadcasted_iota(jnp.int32, (1000, 1000), 0)
    ik_c = jax.lax.broadcasted_iota(jnp.int32, (1000, 1000), 1)
    m1_ref[165:FEAT_DIM, 96:128] = _dn((ik_r == ik_c).astype(f32),
                                       ws_ref[...])

    g70_ref[0:20, :] = _dn(te_ref[...], wp[:, 0:ES])
    g70_ref[20:60, :] = _dn(je_ref[...], wp[:, ES + FD:2 * ES + FD])
    g70_ref[60:70, :] = _dn(tab_ref[...], wp[:, 2 * ES + FD:3 * ES + FD])
    s2r = jax.lax.broadcasted_iota(jnp.int32, (FD, PD), 0)
    s2c = jax.lax.broadcasted_iota(jnp.int32, (FD, PD), 1)
    wp2_ref[...] = _dn((s2c == s2r + ES).astype(f32), wp)
    s4r = jax.lax.broadcasted_iota(jnp.int32, (ES, PD), 0)
    s4c = jax.lax.broadcasted_iota(jnp.int32, (ES, PD), 1)
    wp4 = _dn((s4c == s4r + 2 * ES + FD).astype(f32), wp)
    wp4_ref[...] = wp4
    wp5_ref[...] = _dn((s4c == s4r + PD - ES).astype(f32), wp)
    bp2_ref[...] = bp_ref[...] + jnp.dot(
        bs_ref[...], wp4, preferred_element_type=jnp.float32)


@functools.partial(jax.jit, static_argnames=())
def kernel(feature, typeEmbed, tableEmbed, columnEmbed, opEmbed, joinEmbed,
           Wf, bf, Wf2, bf2, Ws, bs, Wh, bh, Wp, bp):
    b = feature.shape[0]
    f32 = jnp.float32

    full = lambda s: pl.BlockSpec(s, lambda *_: (0,) * len(s))
    shp = lambda *s: jax.ShapeDtypeStruct(s, f32)
    (a102, v3, b1, w2, b2, m1, g70, wp2, wp4, wp5, bp2) = pl.pallas_call(
        _prep_body,
        grid=(1,),
        in_specs=[full(x) for x in
                  ((20, ES), (10, ES), (30, ES), (4, 4), (40, ES),
                   (FD, FD), (1, FD), (FD, FD), (1, FD), (ES, 1000),
                   (1, ES), (ES, BIN), (PD, PD), (1, PD))],
        out_specs=[full(x) for x in
                   ((102, 3 * FD), (3, 3 * FD), (1, 3 * FD),
                    (3 * FD, 3 * FD), (1, 3 * FD), (FEAT_DIM, 128),
                    (70, PD), (FD, PD), (ES, PD), (ES, PD), (1, PD))],
        out_shape=[shp(102, 3 * FD), shp(3, 3 * FD), shp(1, 3 * FD),
                   shp(3 * FD, 3 * FD), shp(1, 3 * FD), shp(FEAT_DIM, 128),
                   shp(70, PD), shp(FD, PD), shp(ES, PD), shp(ES, PD),
                   shp(1, PD)],
    )(typeEmbed, tableEmbed, columnEmbed, opEmbed, joinEmbed,
      Wf, bf[None], Wf2, bf2[None], Ws, bs[None], Wh, Wp, bp[None])
    bh2 = bh[None]

    grid = (b // TILE,)
    full = lambda s: pl.BlockSpec(s, lambda i: (0,) * len(s))
    out = pl.pallas_call(
        _body,
        grid=grid,
        in_specs=[
            pl.BlockSpec((TILE, FEAT_DIM), lambda i: (i, 0)),
            full(a102.shape), full(v3.shape), full(b1.shape),
            full(w2.shape), full(b2.shape), full(m1.shape), full(bh2.shape),
            full(g70.shape), full(wp2.shape), full(wp4.shape),
            full(wp5.shape), full(bp2.shape),
        ],
        out_specs=pl.BlockSpec((TILE, PD), lambda i: (i, 0)),
        out_shape=jax.ShapeDtypeStruct((b, PD), f32),
        compiler_params=pltpu.CompilerParams(
            dimension_semantics=("arbitrary",),
        ),
    )(feature, a102, v3, b1, w2, b2, m1, bh2, g70, wp2, wp4, wp5, bp2)
    return out


# speedup vs baseline: 4.3126x; 1.0096x over previous
import functools

import jax
import jax.numpy as jnp
from jax.experimental import pallas as pl
from jax.experimental.pallas import tpu as pltpu

ES = 32
BIN = 50
FD = ES + ES // 8 + 1
PD = 5 * ES + ES // 8 + 1
FEAT_DIM = 1 + 1 + 9 + 3 + BIN * 3 + 1001

TILE = 512


def _leaky(x):
    return jnp.where(x >= 0, x, 0.01 * x)


def _dn(a, b):
    return jax.lax.dot_general(a, b, (((1,), (1,)), ((), ())),
                               preferred_element_type=jnp.float32)


def _iota2(shape, dim):
    return jax.lax.broadcasted_iota(jnp.int32, shape, dim)


def _body(f_ref, te_ref, tab_ref, ce_ref, oe_ref, je_ref, wf_ref, bf_ref,
          wf2_ref, bf2_ref, ws_ref, bs_ref, wh_ref, bh_ref, wp_ref, bp_ref,
          o_ref,
          a102_s, v3_s, b1_s, w2_s, b2_s, m1_s, g70_s, wp2_s, wp4_s,
          wp5_s, bp2_s):
    f32 = jnp.float32

    @pl.when(pl.program_id(0) == 0)
    def _prep():
        wf = wf_ref[...]
        bf = bf_ref[...][None, :]
        bf2 = bf2_ref[...][None, :]
        wp = wp_ref[...]

        a_col = _dn(ce_ref[...], wf[:, 0:ES])
        a_op = _dn(oe_ref[...], wf[:, ES:ES + 4])
        e36 = (_iota2((1, FD), 1) == FD - 1).astype(f32)
        a_val = _dn(e36, wf)
        a102_s[...] = jnp.zeros((102, 3 * FD), f32)
        a102_s[0:30, 0:FD] = a_col
        a102_s[30:60, FD:2 * FD] = a_col
        a102_s[60:90, 2 * FD:3 * FD] = a_col
        a102_s[90:94, 0:FD] = a_op
        a102_s[94:98, FD:2 * FD] = a_op
        a102_s[98:102, 2 * FD:3 * FD] = a_op
        v3_s[...] = jnp.zeros((3, 3 * FD), f32)
        v3_s[0:1, 0:FD] = a_val
        v3_s[1:2, FD:2 * FD] = a_val
        v3_s[2:3, 2 * FD:3 * FD] = a_val
        b1_s[0:1, 0:FD] = bf
        b1_s[0:1, FD:2 * FD] = bf
        b1_s[0:1, 2 * FD:3 * FD] = bf

        i37 = (_iota2((FD, FD), 0) == _iota2((FD, FD), 1)).astype(f32)
        wf2t = _dn(i37, wf2_ref[...])
        w2_s[...] = jnp.zeros((3 * FD, 3 * FD), f32)
        w2_s[0:FD, 0:FD] = wf2t
        w2_s[FD:2 * FD, FD:2 * FD] = wf2t
        w2_s[2 * FD:3 * FD, 2 * FD:3 * FD] = wf2t
        b2_s[0:1, 0:FD] = bf2
        b2_s[0:1, FD:2 * FD] = bf2
        b2_s[0:1, 2 * FD:3 * FD] = bf2

        m1_s[...] = jnp.zeros((FEAT_DIM, 128), f32)
        rr = _iota2((3 * BIN, BIN), 0)
        cc = _iota2((3 * BIN, BIN), 1)
        wh = wh_ref[...]
        for j in range(3):
            ej = (rr == 3 * cc + j).astype(f32)
            m1_s[14:164, ES * j:ES * (j + 1)] = _dn(ej, wh)
        ik = (_iota2((1000, 1000), 0) == _iota2((1000, 1000), 1)).astype(f32)
        m1_s[165:FEAT_DIM, 96:128] = _dn(ik, ws_ref[...])

        g70_s[0:20, :] = _dn(te_ref[...], wp[:, 0:ES])
        g70_s[20:60, :] = _dn(je_ref[...], wp[:, ES + FD:2 * ES + FD])
        g70_s[60:70, :] = _dn(tab_ref[...], wp[:, 2 * ES + FD:3 * ES + FD])
        s2 = (_iota2((FD, PD), 1) == _iota2((FD, PD), 0) + ES).astype(f32)
        wp2_s[...] = _dn(s2, wp)
        s4 = (_iota2((ES, PD), 1)
              == _iota2((ES, PD), 0) + 2 * ES + FD).astype(f32)
        wp4 = _dn(s4, wp)
        wp4_s[...] = wp4
        s5 = (_iota2((ES, PD), 1) == _iota2((ES, PD), 0) + PD - ES).astype(f32)
        wp5_s[...] = _dn(s5, wp)
        bp2_s[...] = bp_ref[...][None, :] + jnp.dot(
            bs_ref[...][None, :], wp4, preferred_element_type=f32)

    f = f_ref[...]
    t = f.shape[0]

    type_id = f[:, 0:1].astype(jnp.int32)
    join_id = f[:, 1:2].astype(jnp.int32)
    cols = f[:, 2:5].astype(jnp.int32)
    ops = f[:, 5:8].astype(jnp.int32)
    vals = f[:, 8:11]
    m = f[:, 11:14]
    tab_id = f[:, 164:165].astype(jnp.int32)

    i102 = _iota2((t, 102), 1)
    oh102 = (
        (i102 == cols[:, 0:1]).astype(jnp.float32)
        + (i102 == cols[:, 1:2] + 30).astype(jnp.float32)
        + (i102 == cols[:, 2:3] + 60).astype(jnp.float32)
        + (i102 == ops[:, 0:1] + 90).astype(jnp.float32)
        + (i102 == ops[:, 1:2] + 94).astype(jnp.float32)
        + (i102 == ops[:, 2:3] + 98).astype(jnp.float32)
    )
    h1 = (jnp.dot(oh102, a102_s[...], preferred_element_type=jnp.float32)
          + jnp.dot(vals, v3_s[...], preferred_element_type=jnp.float32)
          + b1_s[...])
    h1 = _leaky(h1)
    h2 = _leaky(jnp.dot(h1, w2_s[...], preferred_element_type=jnp.float32)
                + b2_s[...])

    nf = m[:, 0:1] + m[:, 1:2] + m[:, 2:3]
    zero = jnp.zeros_like(h2[:, 0:FD])
    fsum = (jnp.where(m[:, 0:1] != 0, h2[:, 0:FD], zero)
            + jnp.where(m[:, 1:2] != 0, h2[:, FD:2 * FD], zero)
            + jnp.where(m[:, 2:3] != 0, h2[:, 2 * FD:3 * FD], zero))
    filter_emb = fsum / nf

    hs = jnp.dot(f, m1_s[...], preferred_element_type=jnp.float32)
    bh = bh_ref[...][None, :]
    zero32 = jnp.zeros_like(hs[:, 0:ES])
    hist_sum = (jnp.where(m[:, 0:1] != 0, hs[:, 0:ES] + bh, zero32)
                + jnp.where(m[:, 1:2] != 0, hs[:, ES:2 * ES] + bh, zero32)
                + jnp.where(m[:, 2:3] != 0, hs[:, 2 * ES:3 * ES] + bh, zero32))
    hist_emb = hist_sum / nf
    samp = hs[:, 96:128]

    i70 = _iota2((t, 70), 1)
    oh70 = ((i70 == type_id).astype(jnp.float32)
            + (i70 == join_id + 20).astype(jnp.float32)
            + (i70 == tab_id + 60).astype(jnp.float32))
    pre = (jnp.dot(oh70, g70_s[...], preferred_element_type=jnp.float32)
           + jnp.dot(filter_emb, wp2_s[...], preferred_element_type=jnp.float32)
           + jnp.dot(samp, wp4_s[...], preferred_element_type=jnp.float32)
           + jnp.dot(hist_emb, wp5_s[...], preferred_element_type=jnp.float32)
           + bp2_s[...])
    o_ref[...] = _leaky(pre)


@functools.partial(jax.jit, static_argnames=())
def kernel(feature, typeEmbed, tableEmbed, columnEmbed, opEmbed, joinEmbed,
           Wf, bf, Wf2, bf2, Ws, bs, Wh, bh, Wp, bp):
    b = feature.shape[0]
    f32 = jnp.float32

    full = lambda s: pl.BlockSpec(s, lambda i: (0,) * len(s))
    scratch = lambda *s: pltpu.VMEM(s, f32)
    out = pl.pallas_call(
        _body,
        grid=(b // TILE,),
        in_specs=[
            pl.BlockSpec((TILE, FEAT_DIM), lambda i: (i, 0)),
            full((20, ES)), full((10, ES)), full((30, ES)), full((4, 4)),
            full((40, ES)), full((FD, FD)), full((FD,)), full((FD, FD)),
            full((FD,)), full((ES, 1000)), full((ES,)), full((ES, BIN)),
            full((ES,)), full((PD, PD)), full((PD,)),
        ],
        out_specs=pl.BlockSpec((TILE, PD), lambda i: (i, 0)),
        out_shape=jax.ShapeDtypeStruct((b, PD), f32),
        scratch_shapes=[
            scratch(102, 3 * FD), scratch(3, 3 * FD), scratch(1, 3 * FD),
            scratch(3 * FD, 3 * FD), scratch(1, 3 * FD),
            scratch(FEAT_DIM, 128), scratch(70, PD), scratch(FD, PD),
            scratch(ES, PD), scratch(ES, PD), scratch(1, PD),
        ],
        compiler_params=pltpu.CompilerParams(
            dimension_semantics=("arbitrary",),
        ),
    )(feature, typeEmbed, tableEmbed, columnEmbed, opEmbed, joinEmbed,
      Wf, bf, Wf2, bf2, Ws, bs, Wh, bh, Wp, bp)
    return out


# speedup vs baseline: 14.5486x; 3.3735x over previous
import functools

import jax
import jax.numpy as jnp
from jax.experimental import pallas as pl
from jax.experimental.pallas import tpu as pltpu

ES = 32
BIN = 50
FD = ES + ES // 8 + 1
PD = 5 * ES + ES // 8 + 1
FEAT_DIM = 1 + 1 + 9 + 3 + BIN * 3 + 1001

TILE = 512


def _leaky(x):
    return jnp.where(x >= 0, x, 0.01 * x)


def _dn(a, b):
    return jax.lax.dot_general(a, b, (((1,), (1,)), ((), ())),
                               preferred_element_type=jnp.float32)


def _iota2(shape, dim):
    return jax.lax.broadcasted_iota(jnp.int32, shape, dim)


def _eye(n):
    return (_iota2((n, n), 0) == _iota2((n, n), 1)).astype(jnp.float32)


def _dot(a, b):
    return jnp.dot(a, b, preferred_element_type=jnp.float32)


def _col(row, n):
    w = jnp.where(_iota2((n, n), 0) == _iota2((n, n), 1),
                  jnp.broadcast_to(row, (n, n)), 0.0)
    return jnp.sum(w, axis=1, keepdims=True)


def _body(f_ref, te_ref, tab_ref, ce_ref, oe_ref, je_ref, wf_ref, bf_ref,
          wf2_ref, bf2_ref, ws_ref, bs_ref, wh_ref, bh_ref, wp_ref, bp_ref,
          o_ref,
          a102_s, v3a_s, w2a_s, m1_s, g70_s, wp2_s, wp4_s, wp5a_s):
    f32 = jnp.float32

    @pl.when(pl.program_id(0) == 0)
    def _prep():
        wf = wf_ref[...]
        wp = wp_ref[...]

        a_colt = _dn(wf[:, 0:ES], ce_ref[...])
        a_opt = _dn(wf[:, ES:ES + 4], oe_ref[...])
        a102_s[...] = jnp.zeros((3 * FD, 102), f32)
        a102_s[0:FD, 0:30] = a_colt
        a102_s[FD:2 * FD, 30:60] = a_colt
        a102_s[2 * FD:3 * FD, 60:90] = a_colt
        a102_s[0:FD, 90:94] = a_opt
        a102_s[FD:2 * FD, 94:98] = a_opt
        a102_s[2 * FD:3 * FD, 98:102] = a_opt
        a_val = wf[:, FD - 1:FD]
        bf_col = _col(bf_ref[...][None, :], FD)
        v3a_s[...] = jnp.zeros((3 * FD, 4), f32)
        v3a_s[0:FD, 0:1] = a_val
        v3a_s[FD:2 * FD, 1:2] = a_val
        v3a_s[2 * FD:3 * FD, 2:3] = a_val
        v3a_s[0:FD, 3:4] = bf_col
        v3a_s[FD:2 * FD, 3:4] = bf_col
        v3a_s[2 * FD:3 * FD, 3:4] = bf_col

        wf2 = wf2_ref[...]
        bf2_col = _col(bf2_ref[...][None, :], FD)
        w2a_s[...] = jnp.zeros((3 * FD, 3 * FD + 1), f32)
        w2a_s[0:FD, 0:FD] = wf2
        w2a_s[FD:2 * FD, FD:2 * FD] = wf2
        w2a_s[2 * FD:3 * FD, 2 * FD:3 * FD] = wf2
        w2a_s[0:FD, 3 * FD:3 * FD + 1] = bf2_col
        w2a_s[FD:2 * FD, 3 * FD:3 * FD + 1] = bf2_col
        w2a_s[2 * FD:3 * FD, 3 * FD:3 * FD + 1] = bf2_col

        m1_s[...] = jnp.zeros((128, FEAT_DIM + 1), f32)
        rr = _iota2((3 * BIN, BIN), 0)
        cc = _iota2((3 * BIN, BIN), 1)
        wh = wh_ref[...]
        bh_col = _col(bh_ref[...][None, :], ES)
        for j in range(3):
            ej = (rr == 3 * cc + j).astype(f32)
            m1_s[ES * j:ES * (j + 1), 14:164] = _dn(wh, ej)
            m1_s[ES * j:ES * (j + 1), FEAT_DIM:FEAT_DIM + 1] = bh_col
        m1_s[96:128, 165:FEAT_DIM] = ws_ref[...]

        g70_s[:, 0:20] = _dn(wp[:, 0:ES], te_ref[...])
        g70_s[:, 20:60] = _dn(wp[:, ES + FD:2 * ES + FD], je_ref[...])
        g70_s[:, 60:70] = _dn(wp[:, 2 * ES + FD:3 * ES + FD], tab_ref[...])
        wp2_s[...] = wp[:, ES:ES + FD]
        wp4 = wp[:, 2 * ES + FD:3 * ES + FD]
        wp4_s[...] = wp4
        bp2_row = bp_ref[...][None, :] + _dn(bs_ref[...][None, :], wp4)
        wp5a_s[:, 0:ES] = wp[:, PD - ES:PD]
        wp5a_s[:, ES:ES + 1] = _col(bp2_row, PD)

    f = f_ref[...]
    t = f.shape[1]

    type_id = f[0:1, :].astype(jnp.int32)
    join_id = f[1:2, :].astype(jnp.int32)
    cols = f[2:5, :].astype(jnp.int32)
    ops = f[5:8, :].astype(jnp.int32)
    vals = f[8:11, :]
    m = f[11:14, :]
    tab_id = f[164:165, :].astype(jnp.int32)
    ones1 = jnp.ones((1, t), f32)

    i102 = _iota2((102, t), 0)
    oh102 = (
        (i102 == cols[0:1, :]).astype(f32)
        + (i102 == cols[1:2, :] + 30).astype(f32)
        + (i102 == cols[2:3, :] + 60).astype(f32)
        + (i102 == ops[0:1, :] + 90).astype(f32)
        + (i102 == ops[1:2, :] + 94).astype(f32)
        + (i102 == ops[2:3, :] + 98).astype(f32)
    )
    va = jnp.concatenate([vals, ones1], axis=0)
    h1 = _leaky(_dot(a102_s[...], oh102) + _dot(v3a_s[...], va))
    h1a = jnp.concatenate([h1, ones1], axis=0)
    h2 = _leaky(_dot(w2a_s[...], h1a))

    nf = m[0:1, :] + m[1:2, :] + m[2:3, :]
    zero = jnp.zeros_like(h2[0:FD, :])
    fsum = (jnp.where(m[0:1, :] != 0, h2[0:FD, :], zero)
            + jnp.where(m[1:2, :] != 0, h2[FD:2 * FD, :], zero)
            + jnp.where(m[2:3, :] != 0, h2[2 * FD:3 * FD, :], zero))
    filter_emb = fsum / nf

    fa = jnp.concatenate([f, ones1], axis=0)
    hs = _dot(m1_s[...], fa)
    zero32 = jnp.zeros_like(hs[0:ES, :])
    hist_sum = (jnp.where(m[0:1, :] != 0, hs[0:ES, :], zero32)
                + jnp.where(m[1:2, :] != 0, hs[ES:2 * ES, :], zero32)
                + jnp.where(m[2:3, :] != 0, hs[2 * ES:3 * ES, :], zero32))
    hist_emb = hist_sum / nf
    samp = hs[96:128, :]

    i70 = _iota2((70, t), 0)
    oh70 = ((i70 == type_id).astype(f32)
            + (i70 == join_id + 20).astype(f32)
            + (i70 == tab_id + 60).astype(f32))
    ha = jnp.concatenate([hist_emb, ones1], axis=0)
    pre = (_dot(g70_s[...], oh70)
           + _dot(wp2_s[...], filter_emb)
           + _dot(wp4_s[...], samp)
           + _dot(wp5a_s[...], ha))
    o_ref[...] = _leaky(pre)


@functools.partial(jax.jit, static_argnames=())
def kernel(feature, typeEmbed, tableEmbed, columnEmbed, opEmbed, joinEmbed,
           Wf, bf, Wf2, bf2, Ws, bs, Wh, bh, Wp, bp):
    b = feature.shape[0]
    f32 = jnp.float32

    ft = feature.T
    full = lambda s: pl.BlockSpec(s, lambda i: (0,) * len(s))
    scratch = lambda *s: pltpu.VMEM(s, f32)
    out_t = pl.pallas_call(
        _body,
        grid=(b // TILE,),
        in_specs=[
            pl.BlockSpec((FEAT_DIM, TILE), lambda i: (0, i)),
            full((20, ES)), full((10, ES)), full((30, ES)), full((4, 4)),
            full((40, ES)), full((FD, FD)), full((FD,)), full((FD, FD)),
            full((FD,)), full((ES, 1000)), full((ES,)), full((ES, BIN)),
            full((ES,)), full((PD, PD)), full((PD,)),
        ],
        out_specs=pl.BlockSpec((PD, TILE), lambda i: (0, i)),
        out_shape=jax.ShapeDtypeStruct((PD, b), f32),
        scratch_shapes=[
            scratch(3 * FD, 102), scratch(3 * FD, 4),
            scratch(3 * FD, 3 * FD + 1), scratch(128, FEAT_DIM + 1),
            scratch(PD, 70), scratch(PD, FD), scratch(PD, ES),
            scratch(PD, ES + 1),
        ],
        compiler_params=pltpu.CompilerParams(
            dimension_semantics=("arbitrary",),
        ),
    )(ft, typeEmbed, tableEmbed, columnEmbed, opEmbed, joinEmbed,
      Wf, bf, Wf2, bf2, Ws, bs, Wh, bh, Wp, bp)
    return out_t.T


# speedup vs baseline: 18.8156x; 1.2933x over previous
import functools

import jax
import jax.numpy as jnp
from jax.experimental import pallas as pl
from jax.experimental.pallas import tpu as pltpu

ES = 32
BIN = 50
FD = ES + ES // 8 + 1
PD = 5 * ES + ES // 8 + 1
FEAT_DIM = 1 + 1 + 9 + 3 + BIN * 3 + 1001

TILE = 1024


def _leaky(x):
    return jnp.where(x >= 0, x, 0.01 * x)


def _dn(a, b):
    return jax.lax.dot_general(a, b, (((1,), (1,)), ((), ())),
                               preferred_element_type=jnp.float32)


def _iota2(shape, dim):
    return jax.lax.broadcasted_iota(jnp.int32, shape, dim)


def _eye(n):
    return (_iota2((n, n), 0) == _iota2((n, n), 1)).astype(jnp.float32)


def _dot(a, b):
    return jnp.dot(a, b, preferred_element_type=jnp.float32)


def _col(row, n):
    w = jnp.where(_iota2((n, n), 0) == _iota2((n, n), 1),
                  jnp.broadcast_to(row, (n, n)), 0.0)
    return jnp.sum(w, axis=1, keepdims=True)


def _body(f_ref, te_ref, tab_ref, ce_ref, oe_ref, je_ref, wf_ref, bf_ref,
          wf2_ref, bf2_ref, ws_ref, bs_ref, wh_ref, bh_ref, wp_ref, bp_ref,
          o_ref,
          a102_s, v3a_s, w2a_s, m1_s, g70_s, wp2_s, wp4_s, wp5a_s):
    f32 = jnp.float32

    @pl.when(pl.program_id(0) == 0)
    def _prep():
        wf = wf_ref[...]
        wp = wp_ref[...]

        a_colt = _dn(wf[:, 0:ES], ce_ref[...])
        a_opt = _dn(wf[:, ES:ES + 4], oe_ref[...])
        a102_s[...] = jnp.zeros((3 * FD, 102), f32)
        a102_s[0:FD, 0:30] = a_colt
        a102_s[FD:2 * FD, 30:60] = a_colt
        a102_s[2 * FD:3 * FD, 60:90] = a_colt
        a102_s[0:FD, 90:94] = a_opt
        a102_s[FD:2 * FD, 94:98] = a_opt
        a102_s[2 * FD:3 * FD, 98:102] = a_opt
        a_val = wf[:, FD - 1:FD]
        bf_col = _col(bf_ref[...][None, :], FD)
        v3a_s[...] = jnp.zeros((3 * FD, 4), f32)
        v3a_s[0:FD, 0:1] = a_val
        v3a_s[FD:2 * FD, 1:2] = a_val
        v3a_s[2 * FD:3 * FD, 2:3] = a_val
        v3a_s[0:FD, 3:4] = bf_col
        v3a_s[FD:2 * FD, 3:4] = bf_col
        v3a_s[2 * FD:3 * FD, 3:4] = bf_col

        wf2 = wf2_ref[...]
        bf2_col = _col(bf2_ref[...][None, :], FD)
        w2a_s[...] = jnp.zeros((3 * FD, 3 * FD + 1), f32)
        w2a_s[0:FD, 0:FD] = wf2
        w2a_s[FD:2 * FD, FD:2 * FD] = wf2
        w2a_s[2 * FD:3 * FD, 2 * FD:3 * FD] = wf2
        w2a_s[0:FD, 3 * FD:3 * FD + 1] = bf2_col
        w2a_s[FD:2 * FD, 3 * FD:3 * FD + 1] = bf2_col
        w2a_s[2 * FD:3 * FD, 3 * FD:3 * FD + 1] = bf2_col

        m1_s[...] = jnp.zeros((128, FEAT_DIM + 1), f32)
        rr = _iota2((3 * BIN, BIN), 0)
        cc = _iota2((3 * BIN, BIN), 1)
        wh = wh_ref[...]
        bh_col = _col(bh_ref[...][None, :], ES)
        for j in range(3):
            ej = (rr == 3 * cc + j).astype(f32)
            m1_s[ES * j:ES * (j + 1), 14:164] = _dn(wh, ej)
            m1_s[ES * j:ES * (j + 1), FEAT_DIM:FEAT_DIM + 1] = bh_col
        m1_s[96:128, 165:FEAT_DIM] = ws_ref[...]

        g70_s[:, 0:20] = _dn(wp[:, 0:ES], te_ref[...])
        g70_s[:, 20:60] = _dn(wp[:, ES + FD:2 * ES + FD], je_ref[...])
        g70_s[:, 60:70] = _dn(wp[:, 2 * ES + FD:3 * ES + FD], tab_ref[...])
        wp2_s[...] = wp[:, ES:ES + FD]
        wp4 = wp[:, 2 * ES + FD:3 * ES + FD]
        wp4_s[...] = wp4
        bp2_row = bp_ref[...][None, :] + _dn(bs_ref[...][None, :], wp4)
        wp5a_s[:, 0:ES] = wp[:, PD - ES:PD]
        wp5a_s[:, ES:ES + 1] = _col(bp2_row, PD)

    f = f_ref[...]
    t = f.shape[1]

    type_id = f[0:1, :].astype(jnp.int32)
    join_id = f[1:2, :].astype(jnp.int32)
    cols = f[2:5, :].astype(jnp.int32)
    ops = f[5:8, :].astype(jnp.int32)
    vals = f[8:11, :]
    m = f[11:14, :]
    tab_id = f[164:165, :].astype(jnp.int32)
    ones1 = jnp.ones((1, t), f32)

    i102 = _iota2((102, t), 0)
    oh102 = (
        (i102 == cols[0:1, :]).astype(f32)
        + (i102 == cols[1:2, :] + 30).astype(f32)
        + (i102 == cols[2:3, :] + 60).astype(f32)
        + (i102 == ops[0:1, :] + 90).astype(f32)
        + (i102 == ops[1:2, :] + 94).astype(f32)
        + (i102 == ops[2:3, :] + 98).astype(f32)
    )
    va = jnp.concatenate([vals, ones1], axis=0)
    h1 = _leaky(_dot(a102_s[...], oh102) + _dot(v3a_s[...], va))
    h1a = jnp.concatenate([h1, ones1], axis=0)
    h2 = _leaky(_dot(w2a_s[...], h1a))

    nf = m[0:1, :] + m[1:2, :] + m[2:3, :]
    zero = jnp.zeros_like(h2[0:FD, :])
    fsum = (jnp.where(m[0:1, :] != 0, h2[0:FD, :], zero)
            + jnp.where(m[1:2, :] != 0, h2[FD:2 * FD, :], zero)
            + jnp.where(m[2:3, :] != 0, h2[2 * FD:3 * FD, :], zero))
    filter_emb = fsum / nf

    fa = jnp.concatenate([f, ones1], axis=0)
    hs = _dot(m1_s[...], fa)
    zero32 = jnp.zeros_like(hs[0:ES, :])
    hist_sum = (jnp.where(m[0:1, :] != 0, hs[0:ES, :], zero32)
                + jnp.where(m[1:2, :] != 0, hs[ES:2 * ES, :], zero32)
                + jnp.where(m[2:3, :] != 0, hs[2 * ES:3 * ES, :], zero32))
    hist_emb = hist_sum / nf
    samp = hs[96:128, :]

    i70 = _iota2((70, t), 0)
    oh70 = ((i70 == type_id).astype(f32)
            + (i70 == join_id + 20).astype(f32)
            + (i70 == tab_id + 60).astype(f32))
    ha = jnp.concatenate([hist_emb, ones1], axis=0)
    pre = (_dot(g70_s[...], oh70)
           + _dot(wp2_s[...], filter_emb)
           + _dot(wp4_s[...], samp)
           + _dot(wp5a_s[...], ha))
    o_ref[...] = _leaky(pre)


@functools.partial(jax.jit, static_argnames=())
def kernel(feature, typeEmbed, tableEmbed, columnEmbed, opEmbed, joinEmbed,
           Wf, bf, Wf2, bf2, Ws, bs, Wh, bh, Wp, bp):
    b = feature.shape[0]
    f32 = jnp.float32

    ft = feature.T
    full = lambda s: pl.BlockSpec(s, lambda i: (0,) * len(s))
    scratch = lambda *s: pltpu.VMEM(s, f32)
    out_t = pl.pallas_call(
        _body,
        grid=(b // TILE,),
        in_specs=[
            pl.BlockSpec((FEAT_DIM, TILE), lambda i: (0, i)),
            full((20, ES)), full((10, ES)), full((30, ES)), full((4, 4)),
            full((40, ES)), full((FD, FD)), full((FD,)), full((FD, FD)),
            full((FD,)), full((ES, 1000)), full((ES,)), full((ES, BIN)),
            full((ES,)), full((PD, PD)), full((PD,)),
        ],
        out_specs=pl.BlockSpec((PD, TILE), lambda i: (0, i)),
        out_shape=jax.ShapeDtypeStruct((PD, b), f32),
        scratch_shapes=[
            scratch(3 * FD, 102), scratch(3 * FD, 4),
            scratch(3 * FD, 3 * FD + 1), scratch(128, FEAT_DIM + 1),
            scratch(PD, 70), scratch(PD, FD), scratch(PD, ES),
            scratch(PD, ES + 1),
        ],
        compiler_params=pltpu.CompilerParams(
            dimension_semantics=("arbitrary",),
        ),
    )(ft, typeEmbed, tableEmbed, columnEmbed, opEmbed, joinEmbed,
      Wf, bf, Wf2, bf2, Ws, bs, Wh, bh, Wp, bp)
    return out_t.T


# speedup vs baseline: 21.0882x; 1.1208x over previous
import functools

import jax
import jax.numpy as jnp
from jax.experimental import pallas as pl
from jax.experimental.pallas import tpu as pltpu

ES = 32
BIN = 50
FD = ES + ES // 8 + 1
PD = 5 * ES + ES // 8 + 1
FEAT_DIM = 1 + 1 + 9 + 3 + BIN * 3 + 1001

TILE = 2048


def _leaky(x):
    return jnp.where(x >= 0, x, 0.01 * x)


def _dn(a, b):
    return jax.lax.dot_general(a, b, (((1,), (1,)), ((), ())),
                               preferred_element_type=jnp.float32)


def _iota2(shape, dim):
    return jax.lax.broadcasted_iota(jnp.int32, shape, dim)


def _eye(n):
    return (_iota2((n, n), 0) == _iota2((n, n), 1)).astype(jnp.float32)


def _dot(a, b):
    return jnp.dot(a, b, preferred_element_type=jnp.float32)


def _col(row, n):
    w = jnp.where(_iota2((n, n), 0) == _iota2((n, n), 1),
                  jnp.broadcast_to(row, (n, n)), 0.0)
    return jnp.sum(w, axis=1, keepdims=True)


def _body(f_ref, te_ref, tab_ref, ce_ref, oe_ref, je_ref, wf_ref, bf_ref,
          wf2_ref, bf2_ref, ws_ref, bs_ref, wh_ref, bh_ref, wp_ref, bp_ref,
          o_ref,
          a102_s, v3a_s, w2a_s, m1_s, g70_s, wp2_s, wp4_s, wp5a_s):
    f32 = jnp.float32

    @pl.when(pl.program_id(0) == 0)
    def _prep():
        wf = wf_ref[...]
        wp = wp_ref[...]

        a_colt = _dn(wf[:, 0:ES], ce_ref[...])
        a_opt = _dn(wf[:, ES:ES + 4], oe_ref[...])
        a102_s[...] = jnp.zeros((3 * FD, 102), f32)
        a102_s[0:FD, 0:30] = a_colt
        a102_s[FD:2 * FD, 30:60] = a_colt
        a102_s[2 * FD:3 * FD, 60:90] = a_colt
        a102_s[0:FD, 90:94] = a_opt
        a102_s[FD:2 * FD, 94:98] = a_opt
        a102_s[2 * FD:3 * FD, 98:102] = a_opt
        a_val = wf[:, FD - 1:FD]
        bf_col = _col(bf_ref[...][None, :], FD)
        v3a_s[...] = jnp.zeros((3 * FD, 4), f32)
        v3a_s[0:FD, 0:1] = a_val
        v3a_s[FD:2 * FD, 1:2] = a_val
        v3a_s[2 * FD:3 * FD, 2:3] = a_val
        v3a_s[0:FD, 3:4] = bf_col
        v3a_s[FD:2 * FD, 3:4] = bf_col
        v3a_s[2 * FD:3 * FD, 3:4] = bf_col

        wf2 = wf2_ref[...]
        bf2_col = _col(bf2_ref[...][None, :], FD)
        w2a_s[...] = jnp.zeros((3 * FD, 3 * FD + 1), f32)
        w2a_s[0:FD, 0:FD] = wf2
        w2a_s[FD:2 * FD, FD:2 * FD] = wf2
        w2a_s[2 * FD:3 * FD, 2 * FD:3 * FD] = wf2
        w2a_s[0:FD, 3 * FD:3 * FD + 1] = bf2_col
        w2a_s[FD:2 * FD, 3 * FD:3 * FD + 1] = bf2_col
        w2a_s[2 * FD:3 * FD, 3 * FD:3 * FD + 1] = bf2_col

        m1_s[...] = jnp.zeros((128, FEAT_DIM + 1), f32)
        rr = _iota2((3 * BIN, BIN), 0)
        cc = _iota2((3 * BIN, BIN), 1)
        wh = wh_ref[...]
        bh_col = _col(bh_ref[...][None, :], ES)
        for j in range(3):
            ej = (rr == 3 * cc + j).astype(f32)
            m1_s[ES * j:ES * (j + 1), 14:164] = _dn(wh, ej)
            m1_s[ES * j:ES * (j + 1), FEAT_DIM:FEAT_DIM + 1] = bh_col
        m1_s[96:128, 165:FEAT_DIM] = ws_ref[...]

        g70_s[:, 0:20] = _dn(wp[:, 0:ES], te_ref[...])
        g70_s[:, 20:60] = _dn(wp[:, ES + FD:2 * ES + FD], je_ref[...])
        g70_s[:, 60:70] = _dn(wp[:, 2 * ES + FD:3 * ES + FD], tab_ref[...])
        wp2_s[...] = wp[:, ES:ES + FD]
        wp4 = wp[:, 2 * ES + FD:3 * ES + FD]
        wp4_s[...] = wp4
        bp2_row = bp_ref[...][None, :] + _dn(bs_ref[...][None, :], wp4)
        wp5a_s[:, 0:ES] = wp[:, PD - ES:PD]
        wp5a_s[:, ES:ES + 1] = _col(bp2_row, PD)

    f = f_ref[...]
    t = f.shape[1]

    type_id = f[0:1, :].astype(jnp.int32)
    join_id = f[1:2, :].astype(jnp.int32)
    cols = f[2:5, :].astype(jnp.int32)
    ops = f[5:8, :].astype(jnp.int32)
    vals = f[8:11, :]
    m = f[11:14, :]
    tab_id = f[164:165, :].astype(jnp.int32)
    ones1 = jnp.ones((1, t), f32)

    i102 = _iota2((102, t), 0)
    oh102 = (
        (i102 == cols[0:1, :]).astype(f32)
        + (i102 == cols[1:2, :] + 30).astype(f32)
        + (i102 == cols[2:3, :] + 60).astype(f32)
        + (i102 == ops[0:1, :] + 90).astype(f32)
        + (i102 == ops[1:2, :] + 94).astype(f32)
        + (i102 == ops[2:3, :] + 98).astype(f32)
    )
    va = jnp.concatenate([vals, ones1], axis=0)
    h1 = _leaky(_dot(a102_s[...], oh102) + _dot(v3a_s[...], va))
    h1a = jnp.concatenate([h1, ones1], axis=0)
    h2 = _leaky(_dot(w2a_s[...], h1a))

    nf = m[0:1, :] + m[1:2, :] + m[2:3, :]
    zero = jnp.zeros_like(h2[0:FD, :])
    fsum = (jnp.where(m[0:1, :] != 0, h2[0:FD, :], zero)
            + jnp.where(m[1:2, :] != 0, h2[FD:2 * FD, :], zero)
            + jnp.where(m[2:3, :] != 0, h2[2 * FD:3 * FD, :], zero))
    filter_emb = fsum / nf

    fa = jnp.concatenate([f, ones1], axis=0)
    hs = _dot(m1_s[...], fa)
    zero32 = jnp.zeros_like(hs[0:ES, :])
    hist_sum = (jnp.where(m[0:1, :] != 0, hs[0:ES, :], zero32)
                + jnp.where(m[1:2, :] != 0, hs[ES:2 * ES, :], zero32)
                + jnp.where(m[2:3, :] != 0, hs[2 * ES:3 * ES, :], zero32))
    hist_emb = hist_sum / nf
    samp = hs[96:128, :]

    i70 = _iota2((70, t), 0)
    oh70 = ((i70 == type_id).astype(f32)
            + (i70 == join_id + 20).astype(f32)
            + (i70 == tab_id + 60).astype(f32))
    ha = jnp.concatenate([hist_emb, ones1], axis=0)
    pre = (_dot(g70_s[...], oh70)
           + _dot(wp2_s[...], filter_emb)
           + _dot(wp4_s[...], samp)
           + _dot(wp5a_s[...], ha))
    o_ref[...] = _leaky(pre)


@functools.partial(jax.jit, static_argnames=())
def kernel(feature, typeEmbed, tableEmbed, columnEmbed, opEmbed, joinEmbed,
           Wf, bf, Wf2, bf2, Ws, bs, Wh, bh, Wp, bp):
    b = feature.shape[0]
    f32 = jnp.float32

    ft = feature.T
    full = lambda s: pl.BlockSpec(s, lambda i: (0,) * len(s))
    scratch = lambda *s: pltpu.VMEM(s, f32)
    out_t = pl.pallas_call(
        _body,
        grid=(b // TILE,),
        in_specs=[
            pl.BlockSpec((FEAT_DIM, TILE), lambda i: (0, i)),
            full((20, ES)), full((10, ES)), full((30, ES)), full((4, 4)),
            full((40, ES)), full((FD, FD)), full((FD,)), full((FD, FD)),
            full((FD,)), full((ES, 1000)), full((ES,)), full((ES, BIN)),
            full((ES,)), full((PD, PD)), full((PD,)),
        ],
        out_specs=pl.BlockSpec((PD, TILE), lambda i: (0, i)),
        out_shape=jax.ShapeDtypeStruct((PD, b), f32),
        scratch_shapes=[
            scratch(3 * FD, 102), scratch(3 * FD, 4),
            scratch(3 * FD, 3 * FD + 1), scratch(128, FEAT_DIM + 1),
            scratch(PD, 70), scratch(PD, FD), scratch(PD, ES),
            scratch(PD, ES + 1),
        ],
        compiler_params=pltpu.CompilerParams(
            dimension_semantics=("arbitrary",),
        ),
    )(ft, typeEmbed, tableEmbed, columnEmbed, opEmbed, joinEmbed,
      Wf, bf, Wf2, bf2, Ws, bs, Wh, bh, Wp, bp)
    return out_t.T
